# baseline (speedup 1.0000x reference)
import functools

import jax
import jax.numpy as jnp
from jax import lax
from jax.experimental import pallas as pl
from jax.experimental.pallas import tpu as pltpu
from jax.experimental.pallas import tpu_sc as plsc

N_CHUNKS = 100000
D = 512
E = 768
K = 64
TOPD = 8
H = 256
TILE = 4096
COLS = 2048
GRID = (N_CHUNKS + TILE - 1) // TILE

NW = 16
CPW = COLS // NW
VPW = TOPD * CPW
DW = 24
CAND = NW * DW

_HI = lax.Precision.HIGHEST
_BIG = 2**30


def _dot(a, b, prec=_HI):
    return lax.dot_general(a, b, (((1,), (1,)), ((), ())),
                           preferred_element_type=jnp.float32,
                           precision=prec)


def _scan_body(bt_ref, ct_ref, vals_out, flat_out, vals_sc, rows_sc):
    i = pl.program_id(0)

    @pl.when(i == 0)
    def _init():
        vals_sc[...] = jnp.full((TOPD, COLS), -jnp.inf, jnp.float32)
        rows_sc[...] = jnp.zeros((TOPD, COLS), jnp.int32)

    ct = ct_ref[...]
    ct_b = ct.astype(jnp.bfloat16)
    bt_b = bt_ref[...].astype(jnp.bfloat16)
    d = _dot(bt_b, ct_b, prec=None)
    ones = jnp.ones((1, D), dtype=jnp.bfloat16)
    ss = _dot(ones, ct_b * ct_b, prec=None)
    q = d / (jnp.sqrt(ss) + 1e-8)
    col = lax.broadcasted_iota(jnp.int32, (1, TILE), 1)
    gidx = i * TILE + col
    q = jnp.where(gidx < N_CHUNKS, q, -jnp.inf)

    for h in range(TILE // COLS):
        x = q[:, h * COLS:(h + 1) * COLS]
        xr = jnp.full((1, COLS), 2 * i + h, jnp.int32)
        for l in range(TOPD):
            cur = vals_sc[pl.ds(l, 1), :]
            curr = rows_sc[pl.ds(l, 1), :]
            cond = x > cur
            vals_sc[pl.ds(l, 1), :] = jnp.where(cond, x, cur)
            rows_sc[pl.ds(l, 1), :] = jnp.where(cond, xr, curr)
            x = jnp.where(cond, cur, x)
            xr = jnp.where(cond, curr, xr)

    @pl.when(i == GRID - 1)
    def _tail():
        colc = lax.broadcasted_iota(jnp.int32, (TOPD, COLS), 1)
        vals_out[...] = vals_sc[...]
        flat_out[...] = rows_sc[...] * COLS + colc


def _coarse_scan(bt, chunk_traces):
    return pl.pallas_call(
        _scan_body,
        grid=(GRID,),
        in_specs=[
            pl.BlockSpec((1, D), lambda i: (0, 0)),
            pl.BlockSpec((TILE, D), lambda i: (i, 0)),
        ],
        out_specs=[
            pl.BlockSpec((TOPD, COLS), lambda i: (0, 0)),
            pl.BlockSpec((TOPD, COLS), lambda i: (0, 0)),
        ],
        out_shape=[
            jax.ShapeDtypeStruct((TOPD, COLS), jnp.float32),
            jax.ShapeDtypeStruct((TOPD, COLS), jnp.int32),
        ],
        scratch_shapes=[
            pltpu.VMEM((TOPD, COLS), jnp.float32),
            pltpu.VMEM((TOPD, COLS), jnp.int32),
        ],
    )(bt, chunk_traces)


_sc_mesh = plsc.VectorSubcoreMesh(core_axis_name="c", subcore_axis_name="s")


@functools.partial(
    pl.kernel,
    out_type=[
        jax.ShapeDtypeStruct((CAND,), jnp.int32),
        jax.ShapeDtypeStruct((CAND, D), jnp.float32),
        jax.ShapeDtypeStruct((CAND, E), jnp.float32),
    ],
    mesh=_sc_mesh,
    compiler_params=pltpu.CompilerParams(needs_layout_passes=False),
    scratch_types=[
        pltpu.VMEM((VPW,), jnp.float32),
        pltpu.VMEM((VPW,), jnp.int32),
        pltpu.VMEM((DW,), jnp.int32),
        pltpu.VMEM((DW, D), jnp.float32),
        pltpu.VMEM((DW, E), jnp.float32),
        pltpu.SemaphoreType.DMA,
    ],
)
def _sc_select(vals_hbm, flat_hbm, traces_hbm, embs_hbm,
               idx_out, trc_out, emb_out,
               vflat, iflat, idxs, trc_v, emb_v, sem):
    wid = lax.axis_index("s") * 2 + lax.axis_index("c")

    @pl.when(wid < NW)
    def _active():
        lane = lax.iota(jnp.int32, 16)
        ninf = jnp.float32(-jnp.inf)
        NG = VPW // 256

        def vmax16(a):
            for s in (8, 4, 2, 1):
                a = jnp.maximum(a, a.at[(lane + s) & 15]
                                .get(mode="promise_in_bounds"))
            return a

        for r in range(TOPD):
            pltpu.sync_copy(vals_hbm.at[r, pl.ds(wid * CPW, CPW)],
                            vflat.at[pl.ds(r * CPW, CPW)])
            pltpu.sync_copy(flat_hbm.at[r, pl.ds(wid * CPW, CPW)],
                            iflat.at[pl.ds(r * CPW, CPW)])

        p = []
        for g in range(NG):
            pg = jnp.full((16,), ninf, jnp.float32)
            for j in range(16):
                m = vmax16(vflat[pl.ds((g * 16 + j) * 16, 16)])
                pg = jnp.where(lane == j, m, pg)
            p.append(pg)

        def extract(k, p):
            mx = p[0]
            for g in range(1, NG):
                mx = jnp.maximum(mx, p[g])
            gms = vmax16(mx)
            vstar = jnp.full((16,), VPW, jnp.int32)
            for g in range(NG):
                f = plsc.all_reduce_ffs(p[g] == gms)
                vstar = jnp.minimum(
                    vstar, jnp.where(f < 16, f + 16 * g, jnp.int32(VPW)))
            addr = vstar * 16 + lane
            v = plsc.load_gather(vflat, [addr])
            l = plsc.all_reduce_ffs(v == gms)
            fid = plsc.load_gather(iflat, [vstar * 16 + l])
            plsc.store_scatter(idxs, [jnp.full((16,), k, jnp.int32)], fid,
                               mask=lane == 0)
            v2 = jnp.where(lane == l, ninf, v)
            plsc.store_scatter(vflat, [addr], v2)
            nm = vmax16(v2)
            newp = []
            for g in range(NG):
                sel = (vstar >= 16 * g) & (vstar < 16 * (g + 1)) \
                    & (lane == vstar - 16 * g)
                newp.append(jnp.where(sel, nm, p[g]))
            return tuple(newp)

        p = lax.fori_loop(0, DW, extract, tuple(p))

        pltpu.async_copy(traces_hbm.at[idxs], trc_v, sem).wait()
        pltpu.async_copy(embs_hbm.at[idxs], emb_v, sem).wait()
        pltpu.sync_copy(idxs, idx_out.at[pl.ds(wid * DW, DW)])
        pltpu.sync_copy(trc_v, trc_out.at[pl.ds(wid * DW, DW)])
        pltpu.sync_copy(emb_v, emb_out.at[pl.ds(wid * DW, DW)])


def _verify_body(bt_ref, be_ref, idx_ref, trc_ref, emb_ref,
                 w1_ref, b1_ref, w2t_ref, b2_ref, score_out, idx_out):
    trc = trc_ref[...]
    bt = bt_ref[...]
    idxr = idx_ref[...]
    ones = jnp.ones((1, D), dtype=jnp.float32)
    dex = _dot(bt, trc)
    rss = _dot(ones, trc * trc)
    qe = dex / (jnp.sqrt(rss) + 1e-8)

    def sel_body(k, carry):
        qcur, ordv, selv = carry
        m = jnp.max(qcur)
        loc_idx = jnp.min(jnp.where(qcur == m, idxr, _BIG))
        hit = idxr == loc_idx
        ordv = jnp.where(hit, k, ordv)
        selv = jnp.where(hit, 1, selv)
        return jnp.where(hit, -jnp.inf, qcur), ordv, selv

    _, ordv, selv = lax.fori_loop(
        0, K, sel_body,
        (qe, jnp.full((1, CAND), _BIG, jnp.int32),
         jnp.zeros((1, CAND), jnp.int32)))

    w1 = w1_ref[...]
    be = be_ref[...]
    c0 = (lax.dot_general(be, w1[E:2 * E, :], (((1,), (0,)), ((), ())),
                          preferred_element_type=jnp.float32, precision=_HI)
          + lax.dot_general(bt, w1[2 * E + D:, :], (((1,), (0,)), ((), ())),
                            preferred_element_type=jnp.float32, precision=_HI)
          + b1_ref[...])
    h = (lax.dot_general(emb_ref[...], w1[:E, :], (((1,), (0,)), ((), ())),
                         preferred_element_type=jnp.float32, precision=_HI)
         + lax.dot_general(trc, w1[2 * E:2 * E + D, :], (((1,), (0,)), ((), ())),
                           preferred_element_type=jnp.float32, precision=_HI)
         + c0)
    h = jnp.maximum(h, 0.0)
    sc = _dot(w2t_ref[...], h) + b2_ref[0, 0]

    sc_m = jnp.where(selv == 1, sc, -jnp.inf)
    m2 = jnp.max(sc_m)
    loco = jnp.min(jnp.where(sc_m == m2, ordv, _BIG))
    best = jnp.min(jnp.where(ordv == loco, idxr, _BIG))
    score_out[0, 0] = m2
    idx_out[0, 0] = best


def _verify(bt, be, idxs, trc_g, emb_g, W1, b1r, w2t, b2r):
    return pl.pallas_call(
        _verify_body,
        in_specs=[
            pl.BlockSpec((1, D), lambda: (0, 0)),
            pl.BlockSpec((1, E), lambda: (0, 0)),
            pl.BlockSpec((1, CAND), lambda: (0, 0)),
            pl.BlockSpec((CAND, D), lambda: (0, 0)),
            pl.BlockSpec((CAND, E), lambda: (0, 0)),
            pl.BlockSpec((2 * E + 2 * D, H), lambda: (0, 0)),
            pl.BlockSpec((1, H), lambda: (0, 0)),
            pl.BlockSpec((1, H), lambda: (0, 0)),
            pl.BlockSpec((1, 1), lambda: (0, 0),
                         memory_space=pltpu.MemorySpace.SMEM),
        ],
        out_specs=[
            pl.BlockSpec(memory_space=pltpu.MemorySpace.SMEM),
            pl.BlockSpec(memory_space=pltpu.MemorySpace.SMEM),
        ],
        out_shape=[
            jax.ShapeDtypeStruct((1, 1), jnp.float32),
            jax.ShapeDtypeStruct((1, 1), jnp.int32),
        ],
    )(bt, be, idxs, trc_g, emb_g, W1, b1r, w2t, b2r)


@jax.jit
def kernel(backstory_embedding, backstory_trace, chunk_embeddings,
           chunk_traces, W1, b1, W2, b2):
    bt = backstory_trace.reshape(1, D)
    be = backstory_embedding.reshape(1, E)
    b1r = b1.reshape(1, H)
    w2t = W2.reshape(1, H)
    b2r = b2.reshape(1, 1)

    vals, flat = _coarse_scan(bt, chunk_traces)
    idxs, trc_g, emb_g = _sc_select(vals, flat, chunk_traces, chunk_embeddings)
    score, idx = _verify(bt, be, idxs.reshape(1, CAND), trc_g, emb_g,
                         W1, b1r, w2t, b2r)
    return score[0, 0], idx[0, 0]


# speedup vs baseline: 1.0214x; 1.0214x over previous
import functools

import jax
import jax.numpy as jnp
from jax import lax
from jax.experimental import pallas as pl
from jax.experimental.pallas import tpu as pltpu
from jax.experimental.pallas import tpu_sc as plsc

N_CHUNKS = 100000
D = 512
E = 768
K = 64
TOPD = 8
H = 256
TILE = 8192
COLS = 2048
GRID = (N_CHUNKS + TILE - 1) // TILE

NW = 16
CPW = COLS // NW
VPW = TOPD * CPW
DW = 24
CAND = NW * DW

_HI = lax.Precision.HIGHEST
_BIG = 2**30


def _dot(a, b, prec=_HI):
    return lax.dot_general(a, b, (((1,), (1,)), ((), ())),
                           preferred_element_type=jnp.float32,
                           precision=prec)


def _scan_body(bt_ref, ct_ref, vals_out, flat_out, vals_sc, rows_sc):
    i = pl.program_id(0)

    @pl.when(i == 0)
    def _init():
        vals_sc[...] = jnp.full((TOPD, COLS), -jnp.inf, jnp.float32)
        rows_sc[...] = jnp.zeros((TOPD, COLS), jnp.int32)

    ct = ct_ref[...]
    ct_b = ct.astype(jnp.bfloat16)
    bt_b = bt_ref[...].astype(jnp.bfloat16)
    d = _dot(bt_b, ct_b, prec=None)
    ones = jnp.ones((1, D), dtype=jnp.bfloat16)
    ss = _dot(ones, ct_b * ct_b, prec=None)
    q = d / (jnp.sqrt(ss) + 1e-8)
    col = lax.broadcasted_iota(jnp.int32, (1, TILE), 1)
    gidx = i * TILE + col
    q = jnp.where(gidx < N_CHUNKS, q, -jnp.inf)

    for h in range(TILE // COLS):
        x = q[:, h * COLS:(h + 1) * COLS]
        xr = jnp.full((1, COLS), 2 * i + h, jnp.int32)
        for l in range(TOPD):
            cur = vals_sc[pl.ds(l, 1), :]
            curr = rows_sc[pl.ds(l, 1), :]
            cond = x > cur
            vals_sc[pl.ds(l, 1), :] = jnp.where(cond, x, cur)
            rows_sc[pl.ds(l, 1), :] = jnp.where(cond, xr, curr)
            x = jnp.where(cond, cur, x)
            xr = jnp.where(cond, curr, xr)

    @pl.when(i == GRID - 1)
    def _tail():
        colc = lax.broadcasted_iota(jnp.int32, (TOPD, COLS), 1)
        vals_out[...] = vals_sc[...]
        flat_out[...] = rows_sc[...] * COLS + colc


def _coarse_scan(bt, chunk_traces):
    return pl.pallas_call(
        _scan_body,
        grid=(GRID,),
        in_specs=[
            pl.BlockSpec((1, D), lambda i: (0, 0)),
            pl.BlockSpec((TILE, D), lambda i: (i, 0)),
        ],
        out_specs=[
            pl.BlockSpec((TOPD, COLS), lambda i: (0, 0)),
            pl.BlockSpec((TOPD, COLS), lambda i: (0, 0)),
        ],
        out_shape=[
            jax.ShapeDtypeStruct((TOPD, COLS), jnp.float32),
            jax.ShapeDtypeStruct((TOPD, COLS), jnp.int32),
        ],
        scratch_shapes=[
            pltpu.VMEM((TOPD, COLS), jnp.float32),
            pltpu.VMEM((TOPD, COLS), jnp.int32),
        ],
    )(bt, chunk_traces)


_sc_mesh = plsc.VectorSubcoreMesh(core_axis_name="c", subcore_axis_name="s")


@functools.partial(
    pl.kernel,
    out_type=[
        jax.ShapeDtypeStruct((CAND,), jnp.int32),
        jax.ShapeDtypeStruct((CAND, D), jnp.float32),
        jax.ShapeDtypeStruct((CAND, E), jnp.float32),
    ],
    mesh=_sc_mesh,
    compiler_params=pltpu.CompilerParams(needs_layout_passes=False),
    scratch_types=[
        pltpu.VMEM((VPW,), jnp.float32),
        pltpu.VMEM((VPW,), jnp.int32),
        pltpu.VMEM((DW,), jnp.int32),
        pltpu.VMEM((DW, D), jnp.float32),
        pltpu.VMEM((DW, E), jnp.float32),
        pltpu.SemaphoreType.DMA,
    ],
)
def _sc_select(vals_hbm, flat_hbm, traces_hbm, embs_hbm,
               idx_out, trc_out, emb_out,
               vflat, iflat, idxs, trc_v, emb_v, sem):
    wid = lax.axis_index("s") * 2 + lax.axis_index("c")

    @pl.when(wid < NW)
    def _active():
        lane = lax.iota(jnp.int32, 16)
        ninf = jnp.float32(-jnp.inf)
        NG = VPW // 256

        def vmax16(a):
            for s in (8, 4, 2, 1):
                a = jnp.maximum(a, a.at[(lane + s) & 15]
                                .get(mode="promise_in_bounds"))
            return a

        for r in range(TOPD):
            pltpu.sync_copy(vals_hbm.at[r, pl.ds(wid * CPW, CPW)],
                            vflat.at[pl.ds(r * CPW, CPW)])
            pltpu.sync_copy(flat_hbm.at[r, pl.ds(wid * CPW, CPW)],
                            iflat.at[pl.ds(r * CPW, CPW)])

        p = []
        for g in range(NG):
            pg = jnp.full((16,), ninf, jnp.float32)
            for j in range(16):
                m = vmax16(vflat[pl.ds((g * 16 + j) * 16, 16)])
                pg = jnp.where(lane == j, m, pg)
            p.append(pg)

        def extract(k, p):
            mx = p[0]
            for g in range(1, NG):
                mx = jnp.maximum(mx, p[g])
            gms = vmax16(mx)
            vstar = jnp.full((16,), VPW, jnp.int32)
            for g in range(NG):
                f = plsc.all_reduce_ffs(p[g] == gms)
                vstar = jnp.minimum(
                    vstar, jnp.where(f < 16, f + 16 * g, jnp.int32(VPW)))
            addr = vstar * 16 + lane
            v = plsc.load_gather(vflat, [addr])
            l = plsc.all_reduce_ffs(v == gms)
            fid = plsc.load_gather(iflat, [vstar * 16 + l])
            plsc.store_scatter(idxs, [jnp.full((16,), k, jnp.int32)], fid,
                               mask=lane == 0)
            v2 = jnp.where(lane == l, ninf, v)
            plsc.store_scatter(vflat, [addr], v2)
            nm = vmax16(v2)
            newp = []
            for g in range(NG):
                sel = (vstar >= 16 * g) & (vstar < 16 * (g + 1)) \
                    & (lane == vstar - 16 * g)
                newp.append(jnp.where(sel, nm, p[g]))
            return tuple(newp)

        p = lax.fori_loop(0, DW, extract, tuple(p))

        pltpu.async_copy(traces_hbm.at[idxs], trc_v, sem).wait()
        pltpu.async_copy(embs_hbm.at[idxs], emb_v, sem).wait()
        pltpu.sync_copy(idxs, idx_out.at[pl.ds(wid * DW, DW)])
        pltpu.sync_copy(trc_v, trc_out.at[pl.ds(wid * DW, DW)])
        pltpu.sync_copy(emb_v, emb_out.at[pl.ds(wid * DW, DW)])


def _verify_body(bt_ref, be_ref, idx_ref, trc_ref, emb_ref,
                 w1_ref, b1_ref, w2t_ref, b2_ref, score_out, idx_out):
    trc = trc_ref[...]
    bt = bt_ref[...]
    idxr = idx_ref[...]
    ones = jnp.ones((1, D), dtype=jnp.float32)
    dex = _dot(bt, trc)
    rss = _dot(ones, trc * trc)
    qe = dex / (jnp.sqrt(rss) + 1e-8)

    def sel_body(k, carry):
        qcur, ordv, selv = carry
        m = jnp.max(qcur)
        loc_idx = jnp.min(jnp.where(qcur == m, idxr, _BIG))
        hit = idxr == loc_idx
        ordv = jnp.where(hit, k, ordv)
        selv = jnp.where(hit, 1, selv)
        return jnp.where(hit, -jnp.inf, qcur), ordv, selv

    _, ordv, selv = lax.fori_loop(
        0, K, sel_body,
        (qe, jnp.full((1, CAND), _BIG, jnp.int32),
         jnp.zeros((1, CAND), jnp.int32)))

    w1 = w1_ref[...]
    be = be_ref[...]
    c0 = (lax.dot_general(be, w1[E:2 * E, :], (((1,), (0,)), ((), ())),
                          preferred_element_type=jnp.float32, precision=_HI)
          + lax.dot_general(bt, w1[2 * E + D:, :], (((1,), (0,)), ((), ())),
                            preferred_element_type=jnp.float32, precision=_HI)
          + b1_ref[...])
    h = (lax.dot_general(emb_ref[...], w1[:E, :], (((1,), (0,)), ((), ())),
                         preferred_element_type=jnp.float32, precision=_HI)
         + lax.dot_general(trc, w1[2 * E:2 * E + D, :], (((1,), (0,)), ((), ())),
                           preferred_element_type=jnp.float32, precision=_HI)
         + c0)
    h = jnp.maximum(h, 0.0)
    sc = _dot(w2t_ref[...], h) + b2_ref[0, 0]

    sc_m = jnp.where(selv == 1, sc, -jnp.inf)
    m2 = jnp.max(sc_m)
    loco = jnp.min(jnp.where(sc_m == m2, ordv, _BIG))
    best = jnp.min(jnp.where(ordv == loco, idxr, _BIG))
    score_out[0, 0] = m2
    idx_out[0, 0] = best


def _verify(bt, be, idxs, trc_g, emb_g, W1, b1r, w2t, b2r):
    return pl.pallas_call(
        _verify_body,
        in_specs=[
            pl.BlockSpec((1, D), lambda: (0, 0)),
            pl.BlockSpec((1, E), lambda: (0, 0)),
            pl.BlockSpec((1, CAND), lambda: (0, 0)),
            pl.BlockSpec((CAND, D), lambda: (0, 0)),
            pl.BlockSpec((CAND, E), lambda: (0, 0)),
            pl.BlockSpec((2 * E + 2 * D, H), lambda: (0, 0)),
            pl.BlockSpec((1, H), lambda: (0, 0)),
            pl.BlockSpec((1, H), lambda: (0, 0)),
            pl.BlockSpec((1, 1), lambda: (0, 0),
                         memory_space=pltpu.MemorySpace.SMEM),
        ],
        out_specs=[
            pl.BlockSpec(memory_space=pltpu.MemorySpace.SMEM),
            pl.BlockSpec(memory_space=pltpu.MemorySpace.SMEM),
        ],
        out_shape=[
            jax.ShapeDtypeStruct((1, 1), jnp.float32),
            jax.ShapeDtypeStruct((1, 1), jnp.int32),
        ],
    )(bt, be, idxs, trc_g, emb_g, W1, b1r, w2t, b2r)


@jax.jit
def kernel(backstory_embedding, backstory_trace, chunk_embeddings,
           chunk_traces, W1, b1, W2, b2):
    bt = backstory_trace.reshape(1, D)
    be = backstory_embedding.reshape(1, E)
    b1r = b1.reshape(1, H)
    w2t = W2.reshape(1, H)
    b2r = b2.reshape(1, 1)

    vals, flat = _coarse_scan(bt, chunk_traces)
    idxs, trc_g, emb_g = _sc_select(vals, flat, chunk_traces, chunk_embeddings)
    score, idx = _verify(bt, be, idxs.reshape(1, CAND), trc_g, emb_g,
                         W1, b1r, w2t, b2r)
    return score[0, 0], idx[0, 0]


# speedup vs baseline: 1.0269x; 1.0054x over previous
import functools

import jax
import jax.numpy as jnp
from jax import lax
from jax.experimental import pallas as pl
from jax.experimental.pallas import tpu as pltpu
from jax.experimental.pallas import tpu_sc as plsc

N_CHUNKS = 100000
D = 512
E = 768
K = 64
TOPD = 8
H = 256
TILE = 4096
COLS = 2048
GRID = (N_CHUNKS + TILE - 1) // TILE

NW = 16
CPW = COLS // NW
VPW = TOPD * CPW
DW = 24
CAND = NW * DW

_HI = lax.Precision.HIGHEST
_BIG = 2**30


def _dot(a, b, prec=_HI):
    return lax.dot_general(a, b, (((1,), (1,)), ((), ())),
                           preferred_element_type=jnp.float32,
                           precision=prec)


def _scan_body(bt_ref, ct_ref, vals_out, flat_out, vals_sc, rows_sc):
    i = pl.program_id(0)

    @pl.when(i == 0)
    def _init():
        vals_sc[...] = jnp.full((TOPD, COLS), -jnp.inf, jnp.float32)
        rows_sc[...] = jnp.zeros((TOPD, COLS), jnp.int32)

    ct = ct_ref[...]
    ct_b = ct.astype(jnp.bfloat16)
    bt_b = bt_ref[...].astype(jnp.bfloat16)
    d = _dot(bt_b, ct_b, prec=None)
    ones = jnp.ones((1, D), dtype=jnp.bfloat16)
    ss = _dot(ones, ct_b * ct_b, prec=None)
    q = d / (jnp.sqrt(ss) + 1e-8)
    col = lax.broadcasted_iota(jnp.int32, (1, TILE), 1)
    gidx = i * TILE + col
    q = jnp.where(gidx < N_CHUNKS, q, -jnp.inf)

    for h in range(TILE // COLS):
        x = q[:, h * COLS:(h + 1) * COLS]
        xr = jnp.full((1, COLS), 2 * i + h, jnp.int32)
        for l in range(TOPD):
            cur = vals_sc[pl.ds(l, 1), :]
            curr = rows_sc[pl.ds(l, 1), :]
            cond = x > cur
            vals_sc[pl.ds(l, 1), :] = jnp.where(cond, x, cur)
            rows_sc[pl.ds(l, 1), :] = jnp.where(cond, xr, curr)
            x = jnp.where(cond, cur, x)
            xr = jnp.where(cond, curr, xr)

    @pl.when(i == GRID - 1)
    def _tail():
        colc = lax.broadcasted_iota(jnp.int32, (TOPD, COLS), 1)
        vals_out[...] = vals_sc[...]
        flat_out[...] = rows_sc[...] * COLS + colc


def _coarse_scan(bt, chunk_traces):
    return pl.pallas_call(
        _scan_body,
        grid=(GRID,),
        in_specs=[
            pl.BlockSpec((1, D), lambda i: (0, 0)),
            pl.BlockSpec((TILE, D), lambda i: (i, 0)),
        ],
        out_specs=[
            pl.BlockSpec((TOPD, COLS), lambda i: (0, 0)),
            pl.BlockSpec((TOPD, COLS), lambda i: (0, 0)),
        ],
        out_shape=[
            jax.ShapeDtypeStruct((TOPD, COLS), jnp.float32),
            jax.ShapeDtypeStruct((TOPD, COLS), jnp.int32),
        ],
        scratch_shapes=[
            pltpu.VMEM((TOPD, COLS), jnp.float32),
            pltpu.VMEM((TOPD, COLS), jnp.int32),
        ],
    )(bt, chunk_traces)


_sc_mesh = plsc.VectorSubcoreMesh(core_axis_name="c", subcore_axis_name="s")


@functools.partial(
    pl.kernel,
    out_type=[
        jax.ShapeDtypeStruct((CAND,), jnp.int32),
        jax.ShapeDtypeStruct((CAND, D), jnp.float32),
        jax.ShapeDtypeStruct((CAND, E), jnp.float32),
    ],
    mesh=_sc_mesh,
    compiler_params=pltpu.CompilerParams(needs_layout_passes=False),
    scratch_types=[
        pltpu.VMEM((VPW,), jnp.float32),
        pltpu.VMEM((VPW,), jnp.int32),
        pltpu.VMEM((DW,), jnp.int32),
        pltpu.VMEM((DW, D), jnp.float32),
        pltpu.VMEM((DW, E), jnp.float32),
        pltpu.SemaphoreType.DMA,
    ],
)
def _sc_select(vals_hbm, flat_hbm, traces_hbm, embs_hbm,
               idx_out, trc_out, emb_out,
               vflat, iflat, idxs, trc_v, emb_v, sem):
    wid = lax.axis_index("s") * 2 + lax.axis_index("c")

    @pl.when(wid < NW)
    def _active():
        lane = lax.iota(jnp.int32, 16)
        ninf = jnp.float32(-jnp.inf)
        NG = VPW // 256

        def vmax16(a):
            for s in (8, 4, 2, 1):
                a = jnp.maximum(a, a.at[(lane + s) & 15]
                                .get(mode="promise_in_bounds"))
            return a

        for r in range(TOPD):
            pltpu.sync_copy(vals_hbm.at[r, pl.ds(wid * CPW, CPW)],
                            vflat.at[pl.ds(r * CPW, CPW)])
            pltpu.sync_copy(flat_hbm.at[r, pl.ds(wid * CPW, CPW)],
                            iflat.at[pl.ds(r * CPW, CPW)])

        p = []
        for g in range(NG):
            pg = jnp.full((16,), ninf, jnp.float32)
            for j in range(16):
                m = vmax16(vflat[pl.ds((g * 16 + j) * 16, 16)])
                pg = jnp.where(lane == j, m, pg)
            p.append(pg)

        def extract(k, p):
            mx = p[0]
            for g in range(1, NG):
                mx = jnp.maximum(mx, p[g])
            gms = vmax16(mx)
            vstar = jnp.full((16,), VPW, jnp.int32)
            for g in range(NG):
                f = plsc.all_reduce_ffs(p[g] == gms)
                vstar = jnp.minimum(
                    vstar, jnp.where(f < 16, f + 16 * g, jnp.int32(VPW)))
            addr = vstar * 16 + lane
            v = plsc.load_gather(vflat, [addr])
            l = plsc.all_reduce_ffs(v == gms)
            fid = plsc.load_gather(iflat, [vstar * 16 + l])
            plsc.store_scatter(idxs, [jnp.full((16,), k, jnp.int32)], fid,
                               mask=lane == 0)
            v2 = jnp.where(lane == l, ninf, v)
            plsc.store_scatter(vflat, [addr], v2)
            nm = vmax16(v2)
            newp = []
            for g in range(NG):
                sel = (vstar >= 16 * g) & (vstar < 16 * (g + 1)) \
                    & (lane == vstar - 16 * g)
                newp.append(jnp.where(sel, nm, p[g]))
            return tuple(newp)

        p = lax.fori_loop(0, DW, extract, tuple(p))

        pltpu.async_copy(traces_hbm.at[idxs], trc_v, sem).wait()
        pltpu.async_copy(embs_hbm.at[idxs], emb_v, sem).wait()
        pltpu.sync_copy(idxs, idx_out.at[pl.ds(wid * DW, DW)])
        pltpu.sync_copy(trc_v, trc_out.at[pl.ds(wid * DW, DW)])
        pltpu.sync_copy(emb_v, emb_out.at[pl.ds(wid * DW, DW)])


def _verify_body(bt_ref, be_ref, idx_ref, trc_ref, emb_ref,
                 w1_ref, b1_ref, w2t_ref, b2_ref, score_out, idx_out,
                 feats_sc):
    trc = trc_ref[...]
    bt = bt_ref[...]
    idxr = idx_ref[...]
    rss = jnp.sum(trc * trc, axis=1, keepdims=True)
    ctn = trc / (jnp.sqrt(rss) + 1e-8)
    btn = bt / (jnp.sqrt(jnp.sum(bt * bt)) + 1e-8)
    qe = _dot(btn, ctn, prec=None)

    def sel_body(k, carry):
        qcur, ordv, selv = carry
        m = jnp.max(qcur)
        loc_idx = jnp.min(jnp.where(qcur == m, idxr, _BIG))
        hit = idxr == loc_idx
        ordv = jnp.where(hit, k, ordv)
        selv = jnp.where(hit, 1, selv)
        return jnp.where(hit, -jnp.inf, qcur), ordv, selv

    _, ordv, selv = lax.fori_loop(
        0, K, sel_body,
        (qe, jnp.full((1, CAND), _BIG, jnp.int32),
         jnp.zeros((1, CAND), jnp.int32)))

    feats = feats_sc
    feats[:, :E] = emb_ref[...]
    feats[:, E:2 * E] = jnp.broadcast_to(be_ref[...], (CAND, E))
    feats[:, 2 * E:2 * E + D] = trc
    feats[:, 2 * E + D:] = jnp.broadcast_to(bt, (CAND, D))
    h = lax.dot_general(feats[...], w1_ref[...], (((1,), (0,)), ((), ())),
                        preferred_element_type=jnp.float32,
                        precision=None) + b1_ref[...]
    h = jnp.maximum(h, 0.0)
    sc = _dot(w2t_ref[...], h, prec=None) + b2_ref[0, 0]

    sc_m = jnp.where(selv == 1, sc, -jnp.inf)
    m2 = jnp.max(sc_m)
    loco = jnp.min(jnp.where(sc_m == m2, ordv, _BIG))
    best = jnp.min(jnp.where(ordv == loco, idxr, _BIG))
    score_out[0, 0] = m2
    idx_out[0, 0] = best


def _verify(bt, be, idxs, trc_g, emb_g, W1, b1r, w2t, b2r):
    return pl.pallas_call(
        _verify_body,
        in_specs=[
            pl.BlockSpec((1, D), lambda: (0, 0)),
            pl.BlockSpec((1, E), lambda: (0, 0)),
            pl.BlockSpec((1, CAND), lambda: (0, 0)),
            pl.BlockSpec((CAND, D), lambda: (0, 0)),
            pl.BlockSpec((CAND, E), lambda: (0, 0)),
            pl.BlockSpec((2 * E + 2 * D, H), lambda: (0, 0)),
            pl.BlockSpec((1, H), lambda: (0, 0)),
            pl.BlockSpec((1, H), lambda: (0, 0)),
            pl.BlockSpec((1, 1), lambda: (0, 0),
                         memory_space=pltpu.MemorySpace.SMEM),
        ],
        out_specs=[
            pl.BlockSpec(memory_space=pltpu.MemorySpace.SMEM),
            pl.BlockSpec(memory_space=pltpu.MemorySpace.SMEM),
        ],
        out_shape=[
            jax.ShapeDtypeStruct((1, 1), jnp.float32),
            jax.ShapeDtypeStruct((1, 1), jnp.int32),
        ],
        scratch_shapes=[
            pltpu.VMEM((CAND, 2 * E + 2 * D), jnp.float32),
        ],
    )(bt, be, idxs, trc_g, emb_g, W1, b1r, w2t, b2r)


@jax.jit
def kernel(backstory_embedding, backstory_trace, chunk_embeddings,
           chunk_traces, W1, b1, W2, b2):
    bt = backstory_trace.reshape(1, D)
    be = backstory_embedding.reshape(1, E)
    b1r = b1.reshape(1, H)
    w2t = W2.reshape(1, H)
    b2r = b2.reshape(1, 1)

    vals, flat = _coarse_scan(bt, chunk_traces)
    idxs, trc_g, emb_g = _sc_select(vals, flat, chunk_traces, chunk_embeddings)
    score, idx = _verify(bt, be, idxs.reshape(1, CAND), trc_g, emb_g,
                         W1, b1r, w2t, b2r)
    return score[0, 0], idx[0, 0]
